# baseline (speedup 1.0000x reference)
import functools
import jax
import jax.numpy as jnp
from jax import lax
from jax.experimental import pallas as pl
from jax.experimental.pallas import tpu as pltpu
from jax.experimental.pallas import tpu_sc as plsc

D_MODEL = 64
SCALE = 8.0
NC, NS, L = 2, 16, 16
NW = NC * NS
N_I = 4096
N_J = 200
B_TOTAL = N_I * N_J
TC_BLKS = N_I // 128
N_UNITS = N_J * TC_BLKS
U_PER_W = N_UNITS // NW
PITCH = 129

_mesh = plsc.VectorSubcoreMesh(
    core_axis_name="c", subcore_axis_name="s", num_cores=NC, num_subcores=NS
)


@functools.partial(
    pl.kernel,
    out_type=jax.ShapeDtypeStruct((N_J, 8, TC_BLKS, 8, 128), jnp.float32),
    mesh=_mesh,
    scratch_types=[
        pltpu.VMEM((U_PER_W * 128,), jnp.int32),
        pltpu.VMEM((128, D_MODEL), jnp.float32),
        pltpu.VMEM((128, D_MODEL), jnp.float32),
        pltpu.VMEM((128, D_MODEL), jnp.float32),
        pltpu.VMEM((128, D_MODEL), jnp.float32),
        pltpu.VMEM((D_MODEL, PITCH), jnp.float32),
        pltpu.VMEM((D_MODEL, PITCH), jnp.float32),
        pltpu.VMEM((D_MODEL, PITCH), jnp.float32),
        pltpu.VMEM((D_MODEL, PITCH), jnp.float32),
        pltpu.SemaphoreType.DMA((4,)),
        pltpu.SemaphoreType.DMA((4,)),
    ],
    compiler_params=pltpu.CompilerParams(
        use_tc_tiling_on_sc=False, needs_layout_passes=False
    ),
)
def _emb_lookup(idx_hbm, table_hbm, out_hbm, idxw, g0, g1, g2, g3,
                t0, t1, t2, t3, gsem, ssem):
    wid = lax.axis_index("s") * NC + lax.axis_index("c")
    u_base = wid * U_PER_W

    gbuf = (g0, g1, g2, g3)
    tbuf = (t0, t1, t2, t3)
    iota = lax.iota(jnp.int32, L)
    dv = [iota + (k * L) for k in range(4)]

    def prep_and_fire(uu, b):
        pltpu.async_copy(
            table_hbm.at[idxw.at[pl.ds(uu * 128, 128)]], gbuf[b], gsem.at[b]
        )

    def wait_gather(b):
        pltpu.make_async_copy(
            table_hbm.at[pl.ds(0, 128)], gbuf[b], gsem.at[b]
        ).wait()

    def transpose_scale(b):
        @plsc.parallel_loop(0, 128, step=1, unroll=16)
        def row_body(i):
            iv = jnp.full((L,), i, jnp.int32)
            for k in range(4):
                v = gbuf[b][i, pl.ds(k * L, L)] * SCALE
                plsc.store_scatter(tbuf[b], [dv[k], iv], v)

    def fire_store(u, b):
        j = u // TC_BLKS
        tc = u % TC_BLKS
        for dd in range(8):
            pltpu.async_copy(
                tbuf[b].at[pl.ds(dd * 8, 8), pl.ds(0, 128)],
                out_hbm.at[j, dd, tc],
                ssem.at[b],
            )

    def wait_store(b):
        for dd in range(8):
            pltpu.make_async_copy(
                tbuf[b].at[pl.ds(dd * 8, 8), pl.ds(0, 128)],
                out_hbm.at[0, 0, 0],
                ssem.at[b],
            ).wait()

    pltpu.sync_copy(idx_hbm.at[pl.ds(u_base * 128, U_PER_W * 128)], idxw)

    def visit(uu, b, fire_next, drain):
        wait_gather(b)
        if fire_next:
            prep_and_fire(uu + 3, (b + 3) % 4)
        if drain:
            wait_store(b)
        transpose_scale(b)
        fire_store(u_base + uu, b)

    prep_and_fire(0, 0)
    prep_and_fire(1, 1)
    prep_and_fire(2, 2)
    visit(0, 0, True, False)
    visit(1, 1, True, False)
    visit(2, 2, True, False)
    visit(3, 3, True, False)

    def outer_body(o, carry):
        for b in range(4):
            visit(4 * o + b, b, True, True)
        return carry

    lax.fori_loop(1, U_PER_W // 4 - 1, outer_body, 0)

    g0_ = U_PER_W - 4
    visit(g0_ + 0, 0, True, True)
    visit(g0_ + 1, 1, False, True)
    visit(g0_ + 2, 2, False, True)
    visit(g0_ + 3, 3, False, True)
    for b in range(4):
        wait_store(b)


def kernel(x, table):
    idx = x.T.reshape(-1)
    out5 = _emb_lookup(idx, table)
    return out5.transpose(2, 4, 0, 1, 3).reshape(N_I, N_J, D_MODEL)


# speedup vs baseline: 1.0365x; 1.0365x over previous
import functools
import jax
import jax.numpy as jnp
from jax import lax
from jax.experimental import pallas as pl
from jax.experimental.pallas import tpu as pltpu
from jax.experimental.pallas import tpu_sc as plsc

D_MODEL = 64
SCALE = 8.0
NC, NS, L = 2, 16, 16
NW = NC * NS
N_I = 4096
N_J = 200
B_TOTAL = N_I * N_J
TC_BLKS = N_I // 128
N_UNITS = N_J * TC_BLKS
U_PER_W = N_UNITS // NW
PITCH = 129

_mesh = plsc.VectorSubcoreMesh(
    core_axis_name="c", subcore_axis_name="s", num_cores=NC, num_subcores=NS
)


@functools.partial(
    pl.kernel,
    out_type=jax.ShapeDtypeStruct((N_J, 8, TC_BLKS, 8, 128), jnp.float32),
    mesh=_mesh,
    scratch_types=[
        pltpu.VMEM((U_PER_W * 128,), jnp.int32),
        pltpu.VMEM((128, D_MODEL), jnp.float32),
        pltpu.VMEM((128, D_MODEL), jnp.float32),
        pltpu.VMEM((128, D_MODEL), jnp.float32),
        pltpu.VMEM((128, D_MODEL), jnp.float32),
        pltpu.VMEM((D_MODEL, PITCH), jnp.float32),
        pltpu.VMEM((D_MODEL, PITCH), jnp.float32),
        pltpu.VMEM((D_MODEL, PITCH), jnp.float32),
        pltpu.VMEM((D_MODEL, PITCH), jnp.float32),
        pltpu.SemaphoreType.DMA((4,)),
        pltpu.SemaphoreType.DMA((4,)),
    ],
    compiler_params=pltpu.CompilerParams(
        use_tc_tiling_on_sc=False, needs_layout_passes=False
    ),
)
def _emb_lookup(idx_hbm, table_hbm, out_hbm, idxw, g0, g1, g2, g3,
                t0, t1, t2, t3, gsem, ssem):
    wid = lax.axis_index("s") * NC + lax.axis_index("c")
    u_base = wid * U_PER_W

    gbuf = (g0, g1, g2, g3)
    tbuf = (t0, t1, t2, t3)
    iota = lax.iota(jnp.int32, L)
    dv = [iota + (k * L) for k in range(4)]

    def prep_and_fire(uu, b):
        pltpu.async_copy(
            table_hbm.at[idxw.at[pl.ds(uu * 128, 128)]], gbuf[b], gsem.at[b]
        )

    def wait_gather(b):
        pltpu.make_async_copy(
            table_hbm.at[pl.ds(0, 128)], gbuf[b], gsem.at[b]
        ).wait()

    def transpose_scale(b):
        @plsc.parallel_loop(0, 128, step=1, unroll=8)
        def row_body(i):
            iv = jnp.full((L,), i, jnp.int32)
            for k in range(4):
                v = gbuf[b][i, pl.ds(k * L, L)] * SCALE
                plsc.store_scatter(tbuf[b], [dv[k], iv], v)

    def fire_store(u, b):
        j = u // TC_BLKS
        tc = u % TC_BLKS
        for dd in range(8):
            pltpu.async_copy(
                tbuf[b].at[pl.ds(dd * 8, 8), pl.ds(0, 128)],
                out_hbm.at[j, dd, tc],
                ssem.at[b],
            )

    def wait_store(b):
        for dd in range(8):
            pltpu.make_async_copy(
                tbuf[b].at[pl.ds(dd * 8, 8), pl.ds(0, 128)],
                out_hbm.at[0, 0, 0],
                ssem.at[b],
            ).wait()

    pltpu.sync_copy(idx_hbm.at[pl.ds(u_base * 128, U_PER_W * 128)], idxw)

    def visit(uu, b, fire_next, drain):
        wait_gather(b)
        if fire_next:
            prep_and_fire(uu + 3, (b + 3) % 4)
        if drain:
            wait_store(b)
        transpose_scale(b)
        fire_store(u_base + uu, b)

    prep_and_fire(0, 0)
    prep_and_fire(1, 1)
    prep_and_fire(2, 2)
    visit(0, 0, True, False)
    visit(1, 1, True, False)
    visit(2, 2, True, False)
    visit(3, 3, True, False)

    def outer_body(o, carry):
        for b in range(4):
            visit(4 * o + b, b, True, True)
        return carry

    lax.fori_loop(1, U_PER_W // 4 - 1, outer_body, 0)

    g0_ = U_PER_W - 4
    visit(g0_ + 0, 0, True, True)
    visit(g0_ + 1, 1, False, True)
    visit(g0_ + 2, 2, False, True)
    visit(g0_ + 3, 3, False, True)
    for b in range(4):
        wait_store(b)


def kernel(x, table):
    idx = x.T.reshape(-1)
    out5 = _emb_lookup(idx, table)
    return out5.transpose(2, 4, 0, 1, 3).reshape(N_I, N_J, D_MODEL)


# speedup vs baseline: 1.0370x; 1.0005x over previous
import functools
import jax
import jax.numpy as jnp
from jax import lax
from jax.experimental import pallas as pl
from jax.experimental.pallas import tpu as pltpu
from jax.experimental.pallas import tpu_sc as plsc

D_MODEL = 64
SCALE = 8.0
NC, NS, L = 2, 16, 16
NW = NC * NS
N_I = 4096
N_J = 200
B_TOTAL = N_I * N_J
TC_BLKS = N_I // 128
N_UNITS = N_J * TC_BLKS
U_PER_W = N_UNITS // NW
PITCH = 129

_mesh = plsc.VectorSubcoreMesh(
    core_axis_name="c", subcore_axis_name="s", num_cores=NC, num_subcores=NS
)


@functools.partial(
    pl.kernel,
    out_type=jax.ShapeDtypeStruct((N_J, 8, TC_BLKS, 8, 128), jnp.float32),
    mesh=_mesh,
    scratch_types=[
        pltpu.VMEM((U_PER_W * 128,), jnp.int32),
        pltpu.VMEM((128, D_MODEL), jnp.float32),
        pltpu.VMEM((128, D_MODEL), jnp.float32),
        pltpu.VMEM((128, D_MODEL), jnp.float32),
        pltpu.VMEM((128, D_MODEL), jnp.float32),
        pltpu.VMEM((D_MODEL, PITCH), jnp.float32),
        pltpu.VMEM((D_MODEL, PITCH), jnp.float32),
        pltpu.VMEM((D_MODEL, PITCH), jnp.float32),
        pltpu.VMEM((D_MODEL, PITCH), jnp.float32),
        pltpu.SemaphoreType.DMA((4,)),
        pltpu.SemaphoreType.DMA((4,)),
    ],
    compiler_params=pltpu.CompilerParams(
        use_tc_tiling_on_sc=False, needs_layout_passes=False
    ),
)
def _emb_lookup(idx_hbm, table_hbm, out_hbm, idxw, g0, g1, g2, g3,
                t0, t1, t2, t3, gsem, ssem):
    wid = lax.axis_index("s") * NC + lax.axis_index("c")
    u_base = wid * U_PER_W

    gbuf = (g0, g1, g2, g3)
    tbuf = (t0, t1, t2, t3)
    iota = lax.iota(jnp.int32, L)
    dv = [iota + (k * L) for k in range(4)]

    def prep_and_fire(uu, b):
        pltpu.async_copy(
            table_hbm.at[idxw.at[pl.ds(uu * 128, 128)]], gbuf[b], gsem.at[b]
        )

    def wait_gather(b):
        pltpu.make_async_copy(
            table_hbm.at[pl.ds(0, 128)], gbuf[b], gsem.at[b]
        ).wait()

    def transpose_scale(b):
        @plsc.parallel_loop(0, 128, step=2, unroll=4)
        def row_body(i):
            for r in range(2):
                iv = jnp.full((L,), i + r, jnp.int32)
                for k in range(4):
                    v = gbuf[b][i + r, pl.ds(k * L, L)] * SCALE
                    plsc.store_scatter(tbuf[b], [dv[k], iv], v)

    def fire_store(u, b):
        j = u // TC_BLKS
        tc = u % TC_BLKS
        for dd in range(8):
            pltpu.async_copy(
                tbuf[b].at[pl.ds(dd * 8, 8), pl.ds(0, 128)],
                out_hbm.at[j, dd, tc],
                ssem.at[b],
            )

    def wait_store(b):
        for dd in range(8):
            pltpu.make_async_copy(
                tbuf[b].at[pl.ds(dd * 8, 8), pl.ds(0, 128)],
                out_hbm.at[0, 0, 0],
                ssem.at[b],
            ).wait()

    pltpu.sync_copy(idx_hbm.at[pl.ds(u_base * 128, U_PER_W * 128)], idxw)

    def visit(uu, b, fire_next, drain):
        wait_gather(b)
        if fire_next:
            prep_and_fire(uu + 3, (b + 3) % 4)
        if drain:
            wait_store(b)
        transpose_scale(b)
        fire_store(u_base + uu, b)

    prep_and_fire(0, 0)
    prep_and_fire(1, 1)
    prep_and_fire(2, 2)
    visit(0, 0, True, False)
    visit(1, 1, True, False)
    visit(2, 2, True, False)
    visit(3, 3, True, False)

    def outer_body(o, carry):
        for b in range(4):
            visit(4 * o + b, b, True, True)
        return carry

    lax.fori_loop(1, U_PER_W // 4 - 1, outer_body, 0)

    g0_ = U_PER_W - 4
    visit(g0_ + 0, 0, True, True)
    visit(g0_ + 1, 1, False, True)
    visit(g0_ + 2, 2, False, True)
    visit(g0_ + 3, 3, False, True)
    for b in range(4):
        wait_store(b)


def kernel(x, table):
    idx = x.T.reshape(-1)
    out5 = _emb_lookup(idx, table)
    return out5.transpose(2, 4, 0, 1, 3).reshape(N_I, N_J, D_MODEL)
